# baseline (speedup 1.0000x reference)
import jax
import jax.numpy as jnp
from jax.experimental import pallas as pl
from jax.experimental.pallas import tpu as pltpu

S = 100000
B = 1024
D = 64
BETA = 8.0
LR = 0.01
S_BLK = 1024
NS = (S + S_BLK - 1) // S_BLK
INT_MAX = 2147483647


def _em_kernel(q_ref, wv_ref, w_ref, b_ref, wr_ref, k_ref, v_ref,
               attn_ref, retr_ref, energy_ref, nv_ref,
               qp_s, acc_s, l_s, m_s, bi_s):
    p = pl.program_id(0)
    j = pl.program_id(1)

    @pl.when((p == 0) & (j == 0))
    def _init():
        qp_s[...] = jnp.dot(q_ref[...], w_ref[...],
                            preferred_element_type=jnp.float32) + b_ref[...]
        acc_s[...] = jnp.zeros_like(acc_s)
        l_s[...] = jnp.zeros_like(l_s)
        m_s[...] = jnp.full_like(m_s, -jnp.inf)
        bi_s[...] = jnp.zeros_like(bi_s)

    sim = jax.lax.dot_general(qp_s[...], k_ref[...], (((1,), (1,)), ((), ())),
                              preferred_element_type=jnp.float32)
    col = j * S_BLK + jax.lax.broadcasted_iota(jnp.int32, (B, S_BLK), 1)
    sim = jnp.where(col < S, BETA * sim, -jnp.inf)
    e = jnp.exp(sim)

    @pl.when(p == 0)
    def _stats():
        l_s[...] += jnp.sum(e, axis=1, keepdims=True)
        row = j * S_BLK + jax.lax.broadcasted_iota(jnp.int32, (S_BLK, D), 0)
        v = jnp.where(row < S, v_ref[...], 0.0)
        acc_s[...] += jnp.dot(e, v, preferred_element_type=jnp.float32)
        bm = jnp.max(sim, axis=1, keepdims=True)
        bidx = jnp.min(jnp.where(sim == bm, col, INT_MAX), axis=1,
                       keepdims=True)
        upd = bm > m_s[...]
        m_s[...] = jnp.where(upd, bm, m_s[...])
        bi_s[...] = jnp.where(upd, bidx, bi_s[...])

    @pl.when((p == 0) & (j == NS - 1))
    def _finalize():
        retr_ref[...] = acc_s[...] / l_s[...]
        energy_ref[...] = -jnp.log(l_s[...])

    @pl.when(p == 1)
    def _write():
        attn_ref[...] = e / l_s[...]
        eq = bi_s[...] == col
        row_b = jax.lax.broadcasted_iota(jnp.int32, (B, S_BLK), 0)
        winner = jnp.max(jnp.where(eq, row_b, -1), axis=0, keepdims=True)
        oh = jnp.where(eq & (row_b == winner), 1.0, 0.0)
        contrib = jax.lax.dot_general(
            oh, wv_ref[...], (((0,), (0,)), ((), ())),
            preferred_element_type=jnp.float32)
        hasw = jax.lax.dot_general(
            oh, jnp.ones((B, D), jnp.float32), (((0,), (0,)), ((), ())),
            preferred_element_type=jnp.float32)
        newv = v_ref[...] * (1.0 - LR * hasw) + LR * contrib
        nv_ref[...] = jnp.where(wr_ref[0, 0] != 0, newv, v_ref[...])


def kernel(query, write_value, keys, values, W, b, write=1):
    wr = jnp.asarray(write, jnp.int32).reshape(1, 1)
    b2 = jnp.asarray(b, jnp.float32).reshape(1, D)
    attn, retr, energy, nv = pl.pallas_call(
        _em_kernel,
        grid=(2, NS),
        in_specs=[
            pl.BlockSpec((B, D), lambda p, j: (0, 0)),
            pl.BlockSpec((B, D), lambda p, j: (0, 0)),
            pl.BlockSpec((D, D), lambda p, j: (0, 0)),
            pl.BlockSpec((1, D), lambda p, j: (0, 0)),
            pl.BlockSpec((1, 1), lambda p, j: (0, 0)),
            pl.BlockSpec((S_BLK, D), lambda p, j: (j, 0)),
            pl.BlockSpec((S_BLK, D), lambda p, j: (j, 0)),
        ],
        out_specs=[
            pl.BlockSpec((B, S_BLK), lambda p, j: (0, j * p)),
            pl.BlockSpec((B, D), lambda p, j: (0, 0)),
            pl.BlockSpec((B, 1), lambda p, j: (0, 0)),
            pl.BlockSpec((S_BLK, D), lambda p, j: (j * p, 0)),
        ],
        out_shape=[
            jax.ShapeDtypeStruct((B, S), jnp.float32),
            jax.ShapeDtypeStruct((B, D), jnp.float32),
            jax.ShapeDtypeStruct((B, 1), jnp.float32),
            jax.ShapeDtypeStruct((S, D), jnp.float32),
        ],
        scratch_shapes=[
            pltpu.VMEM((B, D), jnp.float32),
            pltpu.VMEM((B, D), jnp.float32),
            pltpu.VMEM((B, 1), jnp.float32),
            pltpu.VMEM((B, 1), jnp.float32),
            pltpu.VMEM((B, 1), jnp.int32),
        ],
        compiler_params=pltpu.CompilerParams(
            dimension_semantics=("arbitrary", "arbitrary"),
        ),
    )(query, write_value, W, b2, wr, keys, values)
    return retr, attn, energy.reshape(B), nv


# speedup vs baseline: 1.0927x; 1.0927x over previous
import jax
import jax.numpy as jnp
from jax.experimental import pallas as pl
from jax.experimental.pallas import tpu as pltpu

S = 100000
B = 1024
D = 64
BETA = 8.0
LR = 0.01
S_BLK = 1024
NS = (S + S_BLK - 1) // S_BLK
S_PAD = NS * S_BLK
NPADCOLS = S_PAD - S
INT_MAX = 2147483647


def _stats_kernel(q_ref, w_ref, b_ref, k_ref, v_ref,
                  retr_ref, energy_ref, bi_ref, invl_ref, qp_ref,
                  acc_s, m_s, bi_s):
    j = pl.program_id(0)

    @pl.when(j == 0)
    def _init():
        qp_ref[...] = BETA * (jnp.dot(q_ref[...], w_ref[...],
                                      preferred_element_type=jnp.float32)
                              + b_ref[...])
        acc_s[...] = jnp.zeros_like(acc_s)
        m_s[...] = jnp.full_like(m_s, -jnp.inf)
        bi_s[...] = jnp.zeros_like(bi_s)

    sim = jax.lax.dot_general(qp_ref[...], k_ref[...],
                              (((1,), (1,)), ((), ())),
                              preferred_element_type=jnp.float32)
    e = jnp.exp(sim)
    vaug = jnp.concatenate(
        [v_ref[...], jnp.ones((S_BLK, 1), jnp.float32)], axis=1)
    acc_s[...] += jnp.dot(e, vaug, preferred_element_type=jnp.float32)
    bm = jnp.max(sim, axis=1, keepdims=True)
    col = j * S_BLK + jax.lax.broadcasted_iota(jnp.int32, (B, S_BLK), 1)
    bidx = jnp.min(jnp.where(sim == bm, col, INT_MAX), axis=1, keepdims=True)
    upd = bm > m_s[...]
    m_s[...] = jnp.where(upd, bm, m_s[...])
    bi_s[...] = jnp.where(upd, bidx, bi_s[...])

    @pl.when(j == NS - 1)
    def _finalize():
        l = acc_s[:, D:D + 1] - float(NPADCOLS)
        retr_ref[...] = acc_s[:, :D] / l
        energy_ref[...] = -jnp.log(l)
        invl_ref[...] = 1.0 / l
        bi_ref[...] = bi_s[...]


def _write_kernel(qp_ref, invl_ref, bic_ref, bir_ref, wv_ref, wr_ref,
                  k_ref, v_ref, attn_ref, nv_ref, keep_s, wvaug_s):
    j = pl.program_id(0)

    @pl.when(j == 0)
    def _init():
        colb = jax.lax.broadcasted_iota(jnp.int32, (B, B), 1)
        win = jnp.max(jnp.where(bir_ref[...] == bic_ref[...], colb, -1),
                      axis=1, keepdims=True)
        rowb = jax.lax.broadcasted_iota(jnp.int32, (B, 1), 0)
        wr_on = wr_ref[0, 0] != 0
        keep_s[...] = jnp.where((win == rowb) & wr_on, 1.0, 0.0)
        wvaug_s[...] = jnp.concatenate(
            [wv_ref[...], jnp.ones((B, 1), jnp.float32)], axis=1)

    sim = jax.lax.dot_general(qp_ref[...], k_ref[...],
                              (((1,), (1,)), ((), ())),
                              preferred_element_type=jnp.float32)
    attn_ref[...] = jnp.exp(sim) * invl_ref[...]

    col = j * S_BLK + jax.lax.broadcasted_iota(jnp.int32, (B, S_BLK), 1)
    eqf = jnp.where(bic_ref[...] == col, keep_s[...], 0.0)
    merged = jax.lax.dot_general(
        eqf, wvaug_s[...], (((0,), (0,)), ((), ())),
        preferred_element_type=jnp.float32)
    hasc = merged[:, D:D + 1]
    nv_ref[...] = v_ref[...] * (1.0 - LR * hasc) + LR * merged[:, :D]


def kernel(query, write_value, keys, values, W, b, write=1):
    b2 = jnp.asarray(b, jnp.float32).reshape(1, D)
    wr = jnp.asarray(write, jnp.int32).reshape(1, 1)
    keys_p = jnp.pad(keys, ((0, NPADCOLS), (0, 0)))
    values_p = jnp.pad(values, ((0, NPADCOLS), (0, 0)))

    retr, energy, bi, invl, qp = pl.pallas_call(
        _stats_kernel,
        grid=(NS,),
        in_specs=[
            pl.BlockSpec((B, D), lambda j: (0, 0)),
            pl.BlockSpec((D, D), lambda j: (0, 0)),
            pl.BlockSpec((1, D), lambda j: (0, 0)),
            pl.BlockSpec((S_BLK, D), lambda j: (j, 0)),
            pl.BlockSpec((S_BLK, D), lambda j: (j, 0)),
        ],
        out_specs=[
            pl.BlockSpec((B, D), lambda j: (0, 0)),
            pl.BlockSpec((B, 1), lambda j: (0, 0)),
            pl.BlockSpec((B, 1), lambda j: (0, 0)),
            pl.BlockSpec((B, 1), lambda j: (0, 0)),
            pl.BlockSpec((B, D), lambda j: (0, 0)),
        ],
        out_shape=[
            jax.ShapeDtypeStruct((B, D), jnp.float32),
            jax.ShapeDtypeStruct((B, 1), jnp.float32),
            jax.ShapeDtypeStruct((B, 1), jnp.int32),
            jax.ShapeDtypeStruct((B, 1), jnp.float32),
            jax.ShapeDtypeStruct((B, D), jnp.float32),
        ],
        scratch_shapes=[
            pltpu.VMEM((B, D + 1), jnp.float32),
            pltpu.VMEM((B, 1), jnp.float32),
            pltpu.VMEM((B, 1), jnp.int32),
        ],
        compiler_params=pltpu.CompilerParams(
            dimension_semantics=("arbitrary",),
        ),
    )(query, W, b2, keys_p, values_p)

    attn, nv = pl.pallas_call(
        _write_kernel,
        grid=(NS,),
        in_specs=[
            pl.BlockSpec((B, D), lambda j: (0, 0)),
            pl.BlockSpec((B, 1), lambda j: (0, 0)),
            pl.BlockSpec((B, 1), lambda j: (0, 0)),
            pl.BlockSpec((1, B), lambda j: (0, 0)),
            pl.BlockSpec((B, D), lambda j: (0, 0)),
            pl.BlockSpec((1, 1), lambda j: (0, 0)),
            pl.BlockSpec((S_BLK, D), lambda j: (j, 0)),
            pl.BlockSpec((S_BLK, D), lambda j: (j, 0)),
        ],
        out_specs=[
            pl.BlockSpec((B, S_BLK), lambda j: (0, j)),
            pl.BlockSpec((S_BLK, D), lambda j: (j, 0)),
        ],
        out_shape=[
            jax.ShapeDtypeStruct((B, S), jnp.float32),
            jax.ShapeDtypeStruct((S, D), jnp.float32),
        ],
        scratch_shapes=[
            pltpu.VMEM((B, 1), jnp.float32),
            pltpu.VMEM((B, D + 1), jnp.float32),
        ],
        compiler_params=pltpu.CompilerParams(
            dimension_semantics=("arbitrary",),
        ),
    )(qp, invl, bi, bi.reshape(1, B), write_value, wr, keys_p, values_p)

    return retr, attn, energy.reshape(B), nv


# speedup vs baseline: 1.1522x; 1.0544x over previous
import jax
import jax.numpy as jnp
from jax.experimental import pallas as pl
from jax.experimental.pallas import tpu as pltpu

S = 100000
B = 1024
D = 64
BETA = 8.0
LR = 0.01
S_BLK = 1024
NS = (S + S_BLK - 1) // S_BLK
NFULL = NS - 1
TAIL = S - NFULL * S_BLK
INT_MAX = 2147483647


def _stats_kernel(q_ref, w_ref, b_ref, k_ref, v_ref, kt_ref, vt_ref,
                  retr_ref, energy_ref, bi_ref, invl_ref, qp_ref,
                  acc_s, m_s, bi_s):
    j = pl.program_id(0)

    @pl.when(j == 0)
    def _init():
        qp_ref[...] = BETA * (jnp.dot(q_ref[...], w_ref[...],
                                      preferred_element_type=jnp.float32)
                              + b_ref[...])
        acc_s[...] = jnp.zeros_like(acc_s)
        m_s[...] = jnp.full_like(m_s, -jnp.inf)
        bi_s[...] = jnp.zeros_like(bi_s)

    iota = jax.lax.broadcasted_iota(jnp.int32, (B, S_BLK), 1)

    sim = jax.lax.dot_general(qp_ref[...], k_ref[...],
                              (((1,), (1,)), ((), ())),
                              preferred_element_type=jnp.float32)
    e = jnp.exp(sim)
    vaug = jnp.concatenate(
        [v_ref[...], jnp.ones((S_BLK, 1), jnp.float32)], axis=1)
    acc_s[...] += jnp.dot(e, vaug, preferred_element_type=jnp.float32)
    bm = jnp.max(sim, axis=1, keepdims=True)
    bidx = j * S_BLK + jnp.min(jnp.where(sim == bm, iota, INT_MAX), axis=1,
                               keepdims=True)
    upd = bm > m_s[...]
    m_s[...] = jnp.where(upd, bm, m_s[...])
    bi_s[...] = jnp.where(upd, bidx, bi_s[...])

    @pl.when(j == NFULL - 1)
    def _tail_and_finalize():
        simt = jax.lax.dot_general(qp_ref[...], kt_ref[...],
                                   (((1,), (1,)), ((), ())),
                                   preferred_element_type=jnp.float32)
        simt = jnp.where(iota < TAIL, simt, -jnp.inf)
        et = jnp.exp(simt)
        rowt = jax.lax.broadcasted_iota(jnp.int32, (S_BLK, D), 0)
        vt = jnp.where(rowt < TAIL, vt_ref[...], 0.0)
        vaugt = jnp.concatenate(
            [vt, jnp.ones((S_BLK, 1), jnp.float32)], axis=1)
        acc = acc_s[...] + jnp.dot(et, vaugt,
                                   preferred_element_type=jnp.float32)
        bmt = jnp.max(simt, axis=1, keepdims=True)
        bidxt = NFULL * S_BLK + jnp.min(
            jnp.where(simt == bmt, iota, INT_MAX), axis=1, keepdims=True)
        updt = bmt > m_s[...]
        bi = jnp.where(updt, bidxt, bi_s[...])

        l = acc[:, D:D + 1]
        retr_ref[...] = acc[:, :D] / l
        energy_ref[...] = -jnp.log(l)
        invl_ref[...] = 1.0 / l
        bi_ref[...] = bi


def _write_kernel(qpa_ref, bic_ref, bir_ref, wv_ref, wr_ref,
                  k_ref, v_ref, attn_ref, nv_ref, keep_s, wvaug_s):
    j = pl.program_id(0)

    @pl.when(j == 0)
    def _init():
        colb = jax.lax.broadcasted_iota(jnp.int32, (B, B), 1)
        win = jnp.max(jnp.where(bir_ref[...] == bic_ref[...], colb, -1),
                      axis=1, keepdims=True)
        rowb = jax.lax.broadcasted_iota(jnp.int32, (B, 1), 0)
        wr_on = wr_ref[0, 0] != 0
        keep_s[...] = jnp.where((win == rowb) & wr_on, 1.0, 0.0)
        wvaug_s[...] = jnp.concatenate(
            [wv_ref[...], jnp.ones((B, 1), jnp.float32)], axis=1)

    kaug = jnp.concatenate(
        [k_ref[...], jnp.ones((S_BLK, 1), jnp.float32)], axis=1)
    sim2 = jax.lax.dot_general(qpa_ref[...], kaug,
                               (((1,), (1,)), ((), ())),
                               preferred_element_type=jnp.float32)
    attn_ref[...] = jnp.exp(sim2)

    iota = jax.lax.broadcasted_iota(jnp.int32, (B, S_BLK), 1)
    eqf = jnp.where(bic_ref[...] - j * S_BLK == iota, keep_s[...], 0.0)
    merged = jax.lax.dot_general(
        eqf, wvaug_s[...], (((0,), (0,)), ((), ())),
        preferred_element_type=jnp.float32)
    hasc = merged[:, D:D + 1]
    nv_ref[...] = v_ref[...] * (1.0 - LR * hasc) + LR * merged[:, :D]


def kernel(query, write_value, keys, values, W, b, write=1):
    b2 = jnp.asarray(b, jnp.float32).reshape(1, D)
    wr = jnp.asarray(write, jnp.int32).reshape(1, 1)

    retr, energy, bi, invl, qp = pl.pallas_call(
        _stats_kernel,
        grid=(NFULL,),
        in_specs=[
            pl.BlockSpec((B, D), lambda j: (0, 0)),
            pl.BlockSpec((D, D), lambda j: (0, 0)),
            pl.BlockSpec((1, D), lambda j: (0, 0)),
            pl.BlockSpec((S_BLK, D), lambda j: (j, 0)),
            pl.BlockSpec((S_BLK, D), lambda j: (j, 0)),
            pl.BlockSpec((S_BLK, D), lambda j: (NFULL, 0)),
            pl.BlockSpec((S_BLK, D), lambda j: (NFULL, 0)),
        ],
        out_specs=[
            pl.BlockSpec((B, D), lambda j: (0, 0)),
            pl.BlockSpec((B, 1), lambda j: (0, 0)),
            pl.BlockSpec((B, 1), lambda j: (0, 0)),
            pl.BlockSpec((B, 1), lambda j: (0, 0)),
            pl.BlockSpec((B, D), lambda j: (0, 0)),
        ],
        out_shape=[
            jax.ShapeDtypeStruct((B, D), jnp.float32),
            jax.ShapeDtypeStruct((B, 1), jnp.float32),
            jax.ShapeDtypeStruct((B, 1), jnp.int32),
            jax.ShapeDtypeStruct((B, 1), jnp.float32),
            jax.ShapeDtypeStruct((B, D), jnp.float32),
        ],
        scratch_shapes=[
            pltpu.VMEM((B, D + 1), jnp.float32),
            pltpu.VMEM((B, 1), jnp.float32),
            pltpu.VMEM((B, 1), jnp.int32),
        ],
        compiler_params=pltpu.CompilerParams(
            dimension_semantics=("arbitrary",),
        ),
    )(query, W, b2, keys, values, keys, values)

    qpa = jnp.concatenate([qp, energy], axis=1)
    attn, nv = pl.pallas_call(
        _write_kernel,
        grid=(NS,),
        in_specs=[
            pl.BlockSpec((B, D + 1), lambda j: (0, 0)),
            pl.BlockSpec((B, 1), lambda j: (0, 0)),
            pl.BlockSpec((1, B), lambda j: (0, 0)),
            pl.BlockSpec((B, D), lambda j: (0, 0)),
            pl.BlockSpec((1, 1), lambda j: (0, 0)),
            pl.BlockSpec((S_BLK, D), lambda j: (j, 0)),
            pl.BlockSpec((S_BLK, D), lambda j: (j, 0)),
        ],
        out_specs=[
            pl.BlockSpec((B, S_BLK), lambda j: (0, j)),
            pl.BlockSpec((S_BLK, D), lambda j: (j, 0)),
        ],
        out_shape=[
            jax.ShapeDtypeStruct((B, S), jnp.float32),
            jax.ShapeDtypeStruct((S, D), jnp.float32),
        ],
        scratch_shapes=[
            pltpu.VMEM((B, 1), jnp.float32),
            pltpu.VMEM((B, D + 1), jnp.float32),
        ],
        compiler_params=pltpu.CompilerParams(
            dimension_semantics=("arbitrary",),
        ),
    )(qpa, bi, bi.reshape(1, B), write_value, wr, keys, values)

    return retr, attn, energy.reshape(B), nv
